# Initial kernel scaffold; baseline (speedup 1.0000x reference)
import jax
import jax.numpy as jnp
from jax.experimental import pallas as pl


def kernel(input_ids, base_table, adapter_A, adapter_B):
    raise NotImplementedError("write your pallas kernel here")


# speedup vs baseline: 9.2819x; 9.2819x over previous
import functools

import jax
import jax.numpy as jnp
from jax import lax
from jax.experimental import pallas as pl
from jax.experimental.pallas import tpu as pltpu
from jax.experimental.pallas import tpu_sc as plsc

D_MODEL = 64
RANK = 32
SCALING = 16.0 / 32.0

try:
    _info = plsc.get_sparse_core_info()
    _NC, _NS = _info.num_cores, _info.num_subcores
except Exception:
    _NC, _NS = 2, 16
_NW = _NC * _NS


def _fuse_body(base_ref, a_ref, b_ref, out_ref):
    out_ref[...] = base_ref[...] + SCALING * jnp.dot(
        a_ref[...], b_ref[...], preferred_element_type=jnp.float32
    )


@functools.lru_cache(maxsize=None)
def _make_fuse(vocab, rank, d_model, blk):
    grid = vocab // blk
    return pl.pallas_call(
        _fuse_body,
        grid=(grid,),
        in_specs=[
            pl.BlockSpec((blk, d_model), lambda i: (i, 0)),
            pl.BlockSpec((blk, rank), lambda i: (i, 0)),
            pl.BlockSpec((rank, d_model), lambda i: (0, 0)),
        ],
        out_specs=pl.BlockSpec((blk, d_model), lambda i: (i, 0)),
        out_shape=jax.ShapeDtypeStruct((vocab, d_model), jnp.float32),
    )


@functools.lru_cache(maxsize=None)
def _make_gather(vocab, d_model, n_tokens, chunk=128):
    n_per_w = n_tokens // _NW
    n_chunks = n_per_w // chunk
    mesh = plsc.VectorSubcoreMesh(core_axis_name="c", subcore_axis_name="s")

    @functools.partial(
        pl.kernel,
        mesh=mesh,
        compiler_params=pltpu.CompilerParams(use_tc_tiling_on_sc=False),
        out_type=jax.ShapeDtypeStruct((n_tokens, d_model), jnp.float32),
        scratch_types=[
            pltpu.VMEM((n_chunks, chunk), jnp.int32),
            pltpu.VMEM((chunk, d_model), jnp.float32),
            pltpu.SemaphoreType.DMA,
        ],
    )
    def gather(table_hbm, idx_hbm, out_hbm, idx_v, rows_v, sem):
        wid = lax.axis_index("s") * _NC + lax.axis_index("c")
        pltpu.sync_copy(idx_hbm.at[wid], idx_v)

        def body(j, carry):
            pltpu.async_copy(table_hbm.at[idx_v.at[j]], rows_v, sem).wait()
            row0 = (wid * n_chunks + j) * chunk
            pltpu.sync_copy(rows_v, out_hbm.at[pl.ds(row0, chunk)])
            return carry

        lax.fori_loop(0, n_chunks, body, 0)

    return gather


def kernel(input_ids, base_table, adapter_A, adapter_B):
    bsz, seq = input_ids.shape
    vocab = base_table.shape[0]
    n_tokens = bsz * seq
    chunk = 128

    fuse = _make_fuse(vocab, RANK, D_MODEL, 4000)
    fused = fuse(base_table, adapter_A, adapter_B)

    n_per_w = n_tokens // _NW
    idx = input_ids.reshape(_NW, n_per_w // chunk, chunk).astype(jnp.int32)
    out = _make_gather(vocab, D_MODEL, n_tokens, chunk)(fused, idx)
    return out.reshape(bsz, seq, D_MODEL)


# speedup vs baseline: 12.3784x; 1.3336x over previous
import functools

import jax
import jax.numpy as jnp
from jax import lax
from jax.experimental import pallas as pl
from jax.experimental.pallas import tpu as pltpu
from jax.experimental.pallas import tpu_sc as plsc

D_MODEL = 64
RANK = 32
SCALING = 16.0 / 32.0

try:
    _info = plsc.get_sparse_core_info()
    _NC, _NS = _info.num_cores, _info.num_subcores
except Exception:
    _NC, _NS = 2, 16
_NW = _NC * _NS


def _fuse_body(base_t_ref, a_t_ref, b_t_ref, out_ref):
    out_ref[...] = base_t_ref[...] + SCALING * jnp.dot(
        b_t_ref[...], a_t_ref[...], preferred_element_type=jnp.float32
    )


@functools.lru_cache(maxsize=None)
def _make_fuse(vocab, rank, d_model, blk):
    grid = (vocab + blk - 1) // blk
    return pl.pallas_call(
        _fuse_body,
        grid=(grid,),
        in_specs=[
            pl.BlockSpec((d_model, blk), lambda i: (0, i)),
            pl.BlockSpec((rank, blk), lambda i: (0, i)),
            pl.BlockSpec((d_model, rank), lambda i: (0, 0)),
        ],
        out_specs=pl.BlockSpec((d_model, blk), lambda i: (0, i)),
        out_shape=jax.ShapeDtypeStruct((d_model, vocab), jnp.float32),
    )


@functools.lru_cache(maxsize=None)
def _make_gather(vocab, d_model, n_tokens, chunk=128):
    n_per_w = n_tokens // _NW
    n_chunks = n_per_w // chunk
    mesh = plsc.VectorSubcoreMesh(core_axis_name="c", subcore_axis_name="s")

    @functools.partial(
        pl.kernel,
        mesh=mesh,
        compiler_params=pltpu.CompilerParams(use_tc_tiling_on_sc=False),
        out_type=jax.ShapeDtypeStruct((n_tokens, d_model), jnp.float32),
        scratch_types=[
            pltpu.VMEM((n_chunks, chunk), jnp.int32),
            pltpu.VMEM((chunk, d_model), jnp.float32),
            pltpu.SemaphoreType.DMA,
        ],
    )
    def gather(table_hbm, idx_hbm, out_hbm, idx_v, rows_v, sem):
        wid = lax.axis_index("s") * _NC + lax.axis_index("c")
        pltpu.sync_copy(idx_hbm.at[wid], idx_v)

        def body(j, carry):
            pltpu.async_copy(table_hbm.at[idx_v.at[j]], rows_v, sem).wait()
            row0 = (wid * n_chunks + j) * chunk
            pltpu.sync_copy(rows_v, out_hbm.at[pl.ds(row0, chunk)])
            return carry

        lax.fori_loop(0, n_chunks, body, 0)

    return gather


def kernel(input_ids, base_table, adapter_A, adapter_B):
    bsz, seq = input_ids.shape
    vocab = base_table.shape[0]
    n_tokens = bsz * seq
    chunk = 128

    fused_t = _make_fuse(vocab, RANK, D_MODEL, 6400)(
        base_table.T, adapter_A.T, adapter_B.T
    )
    fused = jnp.transpose(fused_t)

    n_per_w = n_tokens // _NW
    idx = input_ids.T.reshape(_NW, n_per_w // chunk, chunk).astype(jnp.int32)
    out = _make_gather(vocab, D_MODEL, n_tokens, chunk)(fused, idx)
    return jnp.transpose(out.reshape(seq, bsz, D_MODEL), (1, 0, 2))


# speedup vs baseline: 14.4766x; 1.1695x over previous
import functools

import jax
import jax.numpy as jnp
from jax import lax
from jax.experimental import pallas as pl
from jax.experimental.pallas import tpu as pltpu
from jax.experimental.pallas import tpu_sc as plsc

D_MODEL = 64
RANK = 32
SCALING = 16.0 / 32.0

try:
    _info = plsc.get_sparse_core_info()
    _NC, _NS = _info.num_cores, _info.num_subcores
except Exception:
    _NC, _NS = 2, 16
_NW = _NC * _NS


def _fuse_body(blk, base_t_ref, a_t_ref, b_t_ref, out_ref):
    f_t = base_t_ref[...] + SCALING * jnp.dot(
        b_t_ref[...], a_t_ref[...], preferred_element_type=jnp.float32
    )
    out_ref[...] = jnp.transpose(f_t)


@functools.lru_cache(maxsize=None)
def _make_fuse(vocab, rank, d_model, blk):
    grid = (vocab + blk - 1) // blk
    return pl.pallas_call(
        functools.partial(_fuse_body, blk),
        grid=(grid,),
        in_specs=[
            pl.BlockSpec((d_model, blk), lambda i: (0, i)),
            pl.BlockSpec((rank, blk), lambda i: (0, i)),
            pl.BlockSpec((d_model, rank), lambda i: (0, 0)),
        ],
        out_specs=pl.BlockSpec((blk, d_model), lambda i: (i, 0)),
        out_shape=jax.ShapeDtypeStruct((vocab, d_model), jnp.float32),
    )


@functools.lru_cache(maxsize=None)
def _make_gather(vocab, d_model, n_tokens, chunk=128):
    n_per_w = n_tokens // _NW
    n_chunks = n_per_w // chunk
    mesh = plsc.VectorSubcoreMesh(core_axis_name="c", subcore_axis_name="s")

    @functools.partial(
        pl.kernel,
        mesh=mesh,
        compiler_params=pltpu.CompilerParams(use_tc_tiling_on_sc=False),
        out_type=jax.ShapeDtypeStruct((n_tokens, d_model), jnp.float32),
        scratch_types=[
            pltpu.VMEM((n_chunks, chunk), jnp.int32),
            pltpu.VMEM((chunk, d_model), jnp.float32),
            pltpu.SemaphoreType.DMA,
        ],
    )
    def gather(table_hbm, idx_hbm, out_hbm, idx_v, rows_v, sem):
        wid = lax.axis_index("s") * _NC + lax.axis_index("c")
        pltpu.sync_copy(idx_hbm.at[wid], idx_v)

        def body(j, carry):
            pltpu.async_copy(table_hbm.at[idx_v.at[j]], rows_v, sem).wait()
            row0 = (wid * n_chunks + j) * chunk
            pltpu.sync_copy(rows_v, out_hbm.at[pl.ds(row0, chunk)])
            return carry

        lax.fori_loop(0, n_chunks, body, 0)

    return gather


def kernel(input_ids, base_table, adapter_A, adapter_B):
    bsz, seq = input_ids.shape
    vocab = base_table.shape[0]
    n_tokens = bsz * seq
    chunk = 128

    fused = _make_fuse(vocab, RANK, D_MODEL, 6400)(
        base_table.T, adapter_A.T, adapter_B.T
    )

    n_per_w = n_tokens // _NW
    idx = input_ids.T.reshape(_NW, n_per_w // chunk, chunk).astype(jnp.int32)
    out = _make_gather(vocab, D_MODEL, n_tokens, chunk)(fused, idx)
    return jnp.transpose(out.reshape(seq, bsz, D_MODEL), (1, 0, 2))


# speedup vs baseline: 19.5670x; 1.3516x over previous
import functools

import jax
import jax.numpy as jnp
from jax import lax
from jax.experimental import pallas as pl
from jax.experimental.pallas import tpu as pltpu
from jax.experimental.pallas import tpu_sc as plsc

D_MODEL = 64
RANK = 32
SCALING = 16.0 / 32.0

try:
    _info = plsc.get_sparse_core_info()
    _NC, _NS = _info.num_cores, _info.num_subcores
except Exception:
    _NC, _NS = 2, 16
_NW = _NC * _NS


def _fuse_body(blk, base_t_ref, a_t_ref, b_t_ref, out_ref):
    f_t = base_t_ref[...] + SCALING * jnp.dot(
        b_t_ref[...], a_t_ref[...], preferred_element_type=jnp.float32
    )
    out_ref[:, 0:64] = jnp.transpose(f_t)


@functools.lru_cache(maxsize=None)
def _make_fuse(vocab, rank, d_model, blk):
    grid = (vocab + blk - 1) // blk
    return pl.pallas_call(
        functools.partial(_fuse_body, blk),
        grid=(grid,),
        in_specs=[
            pl.BlockSpec((d_model, blk), lambda i: (0, i)),
            pl.BlockSpec((rank, blk), lambda i: (0, i)),
            pl.BlockSpec((d_model, rank), lambda i: (0, 0)),
        ],
        out_specs=pl.BlockSpec((blk, 2 * d_model), lambda i: (i, 0)),
        out_shape=jax.ShapeDtypeStruct((vocab, 2 * d_model), jnp.float32),
    )


@functools.lru_cache(maxsize=None)
def _make_gather(vocab, d_model, n_tokens, chunk=128):
    n_per_w = n_tokens // _NW
    n_chunks = n_per_w // chunk
    mesh = plsc.VectorSubcoreMesh(core_axis_name="c", subcore_axis_name="s")

    @functools.partial(
        pl.kernel,
        mesh=mesh,
        compiler_params=pltpu.CompilerParams(use_tc_tiling_on_sc=False),
        out_type=jax.ShapeDtypeStruct((n_tokens, d_model), jnp.float32),
        scratch_types=[
            pltpu.VMEM((n_chunks, chunk), jnp.int32),
            pltpu.VMEM((chunk, d_model), jnp.float32),
            pltpu.SemaphoreType.DMA,
        ],
    )
    def gather(table_hbm, idx_hbm, out_hbm, idx_v, rows_v, sem):
        wid = lax.axis_index("s") * _NC + lax.axis_index("c")
        pltpu.sync_copy(idx_hbm.at[wid], idx_v)

        def body(j, carry):
            pltpu.async_copy(table_hbm.at[idx_v.at[j]], rows_v, sem).wait()
            row0 = (wid * n_chunks + j) * chunk
            pltpu.sync_copy(rows_v, out_hbm.at[pl.ds(row0, chunk)])
            return carry

        lax.fori_loop(0, n_chunks, body, 0)

    return gather


def kernel(input_ids, base_table, adapter_A, adapter_B):
    bsz, seq = input_ids.shape
    vocab = base_table.shape[0]
    n_tokens = bsz * seq
    chunk = 128

    fused128 = _make_fuse(vocab, RANK, D_MODEL, 6400)(
        base_table.T, adapter_A.T, adapter_B.T
    )
    fused = fused128.reshape(2 * vocab, D_MODEL)

    n_per_w = n_tokens // _NW
    idx = (input_ids.astype(jnp.int32) * 2).T.reshape(
        _NW, n_per_w // chunk, chunk
    )
    out = _make_gather(2 * vocab, D_MODEL, n_tokens, chunk)(fused, idx)
    return jnp.transpose(out.reshape(seq, bsz, D_MODEL), (1, 0, 2))
